# baseline (device time: 157210 ns/iter reference)
import functools

import jax
import jax.numpy as jnp
from jax import lax
from jax.experimental import pallas as pl
from jax.experimental.pallas import tpu as pltpu

N_DEV = 32


def kernel(q, k, v):
    s_per, d = q.shape

    def body(q_ref, k_ref, v_ref, out_ref, kv_buf, k_full, v_full,
             send_sems, recv_sems):
        my = lax.axis_index("i")
        left = (my - 1) % N_DEV
        right = (my + 1) % N_DEV

        barrier_sem = pltpu.get_barrier_semaphore()
        for nbr in [left, right]:
            pl.semaphore_signal(
                barrier_sem, inc=1,
                device_id=(nbr,), device_id_type=pl.DeviceIdType.MESH,
            )
        pl.semaphore_wait(barrier_sem, 2)

        kv_buf[0, 0, :, :] = k_ref[:, :]
        kv_buf[0, 1, :, :] = v_ref[:, :]
        k_full[pl.ds(my * s_per, s_per), :] = k_ref[:, :]
        v_full[pl.ds(my * s_per, s_per), :] = v_ref[:, :]

        for h in range(N_DEV - 1):
            send_slot = h % 2
            recv_slot = (h + 1) % 2
            rdma = pltpu.make_async_remote_copy(
                src_ref=kv_buf.at[send_slot],
                dst_ref=kv_buf.at[recv_slot],
                send_sem=send_sems.at[send_slot],
                recv_sem=recv_sems.at[recv_slot],
                device_id=(right,),
                device_id_type=pl.DeviceIdType.MESH,
            )
            rdma.start()
            rdma.wait()

            origin = (my - h - 1) % N_DEV
            k_full[pl.ds(origin * s_per, s_per), :] = kv_buf[recv_slot, 0, :, :]
            v_full[pl.ds(origin * s_per, s_per), :] = kv_buf[recv_slot, 1, :, :]

        qv = q_ref[:, :]
        scores = lax.dot_general(
            qv, k_full[:, :],
            dimension_numbers=(((1,), (1,)), ((), ())),
            preferred_element_type=jnp.float32,
        ) * (1.0 / (d ** 0.5))
        m = jnp.max(scores, axis=1, keepdims=True)
        w = jnp.exp(scores - m)
        denom = jnp.sum(w, axis=1, keepdims=True)
        out = lax.dot_general(
            w, v_full[:, :],
            dimension_numbers=(((1,), (0,)), ((), ())),
            preferred_element_type=jnp.float32,
        )
        out_ref[:, :] = out / denom

    return pl.pallas_call(
        body,
        out_shape=jax.ShapeDtypeStruct((s_per, d), jnp.float32),
        in_specs=[
            pl.BlockSpec(memory_space=pltpu.VMEM),
            pl.BlockSpec(memory_space=pltpu.VMEM),
            pl.BlockSpec(memory_space=pltpu.VMEM),
        ],
        out_specs=pl.BlockSpec(memory_space=pltpu.VMEM),
        scratch_shapes=[
            pltpu.VMEM((2, 2, s_per, d), jnp.float32),
            pltpu.VMEM((N_DEV * s_per, d), jnp.float32),
            pltpu.VMEM((N_DEV * s_per, d), jnp.float32),
            pltpu.SemaphoreType.DMA((2,)),
            pltpu.SemaphoreType.DMA((2,)),
        ],
        compiler_params=pltpu.CompilerParams(collective_id=0),
    )(q, k, v)


# device time: 126650 ns/iter; 1.2413x vs baseline; 1.2413x over previous
import functools

import jax
import jax.numpy as jnp
from jax import lax
from jax.experimental import pallas as pl
from jax.experimental.pallas import tpu as pltpu

N_DEV = 32


def kernel(q, k, v):
    s_per, d = q.shape

    def body(q_ref, k_ref, v_ref, out_ref, rbuf, lbuf, k_full, v_full,
             send_sems_r, recv_sems_r, send_sems_l, recv_sems_l):
        my = lax.axis_index("i")
        left = (my - 1) % N_DEV
        right = (my + 1) % N_DEV

        barrier_sem = pltpu.get_barrier_semaphore()
        for nbr in [left, right]:
            pl.semaphore_signal(
                barrier_sem, inc=1,
                device_id=(nbr,), device_id_type=pl.DeviceIdType.MESH,
            )
        pl.semaphore_wait(barrier_sem, 2)

        rbuf[0, 0, :, :] = k_ref[:, :]
        rbuf[0, 1, :, :] = v_ref[:, :]
        lbuf[0, 0, :, :] = k_ref[:, :]
        lbuf[0, 1, :, :] = v_ref[:, :]
        k_full[pl.ds(my * s_per, s_per), :] = k_ref[:, :]
        v_full[pl.ds(my * s_per, s_per), :] = v_ref[:, :]

        HOPS_R = N_DEV // 2
        HOPS_L = N_DEV - 1 - HOPS_R
        for h in range(HOPS_R):
            send_slot = h % 2
            recv_slot = (h + 1) % 2
            rdma_r = pltpu.make_async_remote_copy(
                src_ref=rbuf.at[send_slot],
                dst_ref=rbuf.at[recv_slot],
                send_sem=send_sems_r.at[send_slot],
                recv_sem=recv_sems_r.at[recv_slot],
                device_id=(right,),
                device_id_type=pl.DeviceIdType.MESH,
            )
            rdma_r.start()
            if h < HOPS_L:
                rdma_l = pltpu.make_async_remote_copy(
                    src_ref=lbuf.at[send_slot],
                    dst_ref=lbuf.at[recv_slot],
                    send_sem=send_sems_l.at[send_slot],
                    recv_sem=recv_sems_l.at[recv_slot],
                    device_id=(left,),
                    device_id_type=pl.DeviceIdType.MESH,
                )
                rdma_l.start()

            rdma_r.wait()
            origin_r = (my - h - 1) % N_DEV
            k_full[pl.ds(origin_r * s_per, s_per), :] = rbuf[recv_slot, 0, :, :]
            v_full[pl.ds(origin_r * s_per, s_per), :] = rbuf[recv_slot, 1, :, :]

            if h < HOPS_L:
                rdma_l.wait()
                origin_l = (my + h + 1) % N_DEV
                k_full[pl.ds(origin_l * s_per, s_per), :] = lbuf[recv_slot, 0, :, :]
                v_full[pl.ds(origin_l * s_per, s_per), :] = lbuf[recv_slot, 1, :, :]

        qv = q_ref[:, :]
        scores = lax.dot_general(
            qv, k_full[:, :],
            dimension_numbers=(((1,), (1,)), ((), ())),
            preferred_element_type=jnp.float32,
        ) * (1.0 / (d ** 0.5))
        m = jnp.max(scores, axis=1, keepdims=True)
        w = jnp.exp(scores - m)
        denom = jnp.sum(w, axis=1, keepdims=True)
        out = lax.dot_general(
            w, v_full[:, :],
            dimension_numbers=(((1,), (0,)), ((), ())),
            preferred_element_type=jnp.float32,
        )
        out_ref[:, :] = out / denom

    return pl.pallas_call(
        body,
        out_shape=jax.ShapeDtypeStruct((s_per, d), jnp.float32),
        in_specs=[
            pl.BlockSpec(memory_space=pltpu.VMEM),
            pl.BlockSpec(memory_space=pltpu.VMEM),
            pl.BlockSpec(memory_space=pltpu.VMEM),
        ],
        out_specs=pl.BlockSpec(memory_space=pltpu.VMEM),
        scratch_shapes=[
            pltpu.VMEM((2, 2, s_per, d), jnp.float32),
            pltpu.VMEM((2, 2, s_per, d), jnp.float32),
            pltpu.VMEM((N_DEV * s_per, d), jnp.float32),
            pltpu.VMEM((N_DEV * s_per, d), jnp.float32),
            pltpu.SemaphoreType.DMA((2,)),
            pltpu.SemaphoreType.DMA((2,)),
            pltpu.SemaphoreType.DMA((2,)),
            pltpu.SemaphoreType.DMA((2,)),
        ],
        compiler_params=pltpu.CompilerParams(collective_id=0),
    )(q, k, v)


# device time: 62310 ns/iter; 2.5230x vs baseline; 2.0326x over previous
import jax
import jax.numpy as jnp
from jax import lax
from jax.experimental import pallas as pl
from jax.experimental.pallas import tpu as pltpu

N_DEV = 32
ZG = 4
PG = 8


def kernel(q, k, v):
    s_per, d = q.shape
    scale = 1.0 / (d ** 0.5)

    def body(q_ref, k_ref, v_ref, out_ref, pblocks,
             z_send, z_recv, p_send, p_recv):
        my = lax.axis_index("i")
        j = my % PG
        base = my - j

        barrier_sem = pltpu.get_barrier_semaphore()
        n_peers = 0
        for dz in range(1, ZG):
            pl.semaphore_signal(
                barrier_sem, inc=1,
                device_id=((my + PG * dz) % N_DEV,),
                device_id_type=pl.DeviceIdType.MESH,
            )
            n_peers += 1
        for o in range(1, PG):
            pl.semaphore_signal(
                barrier_sem, inc=1,
                device_id=(base + (j + o) % PG,),
                device_id_type=pl.DeviceIdType.MESH,
            )
            n_peers += 1
        pl.semaphore_wait(barrier_sem, n_peers)

        pblocks[0, 0, 0, :, :] = k_ref[:, :].astype(jnp.bfloat16)
        pblocks[0, 0, 1, :, :] = v_ref[:, :].astype(jnp.bfloat16)

        z_rdmas = []
        for dz in range(1, ZG):
            r = pltpu.make_async_remote_copy(
                src_ref=pblocks.at[0, 0],
                dst_ref=pblocks.at[0, ZG - dz],
                send_sem=z_send.at[dz],
                recv_sem=z_recv.at[ZG - dz],
                device_id=((my + PG * dz) % N_DEV,),
                device_id_type=pl.DeviceIdType.MESH,
            )
            r.start()
            z_rdmas.append(r)

        qv = q_ref[:, :]
        q_bf = qv.astype(jnp.bfloat16)

        def flash_update(m, l, acc, kc, vc):
            s_blk = lax.dot_general(
                q_bf, kc, dimension_numbers=(((1,), (1,)), ((), ())),
                preferred_element_type=jnp.float32,
            ) * scale
            m_c = jnp.max(s_blk, axis=1, keepdims=True)
            m_new = jnp.maximum(m, m_c)
            alpha = jnp.exp(m - m_new)
            p = jnp.exp(s_blk - m_new)
            l_new = l * alpha + jnp.sum(p, axis=1, keepdims=True)
            acc_new = acc * alpha + lax.dot_general(
                p.astype(jnp.bfloat16), vc,
                dimension_numbers=(((1,), (0,)), ((), ())),
                preferred_element_type=jnp.float32,
            )
            return m_new, l_new, acc_new

        s0 = lax.dot_general(
            qv, k_ref[:, :], dimension_numbers=(((1,), (1,)), ((), ())),
            preferred_element_type=jnp.float32,
        ) * scale
        m = jnp.max(s0, axis=1, keepdims=True)
        p0 = jnp.exp(s0 - m)
        l = jnp.sum(p0, axis=1, keepdims=True)
        acc = lax.dot_general(
            p0, v_ref[:, :], dimension_numbers=(((1,), (0,)), ((), ())),
            preferred_element_type=jnp.float32,
        )

        for dz in range(1, ZG):
            z_rdmas[dz - 1].wait_recv()
            i = ZG - dz
            m, l, acc = flash_update(m, l, acc, pblocks[0, i, 0, :, :],
                                     pblocks[0, i, 1, :, :])

        p_rdmas = []
        for o in range(1, PG):
            r = pltpu.make_async_remote_copy(
                src_ref=pblocks.at[0],
                dst_ref=pblocks.at[PG - o],
                send_sem=p_send.at[o],
                recv_sem=p_recv.at[PG - o],
                device_id=(base + (j + o) % PG,),
                device_id_type=pl.DeviceIdType.MESH,
            )
            r.start()
            p_rdmas.append(r)

        for o in range(1, PG):
            p_rdmas[o - 1].wait_recv()
            s = PG - o
            kc = jnp.reshape(pblocks[s, :, 0, :, :], (ZG * s_per, d))
            vc = jnp.reshape(pblocks[s, :, 1, :, :], (ZG * s_per, d))
            m, l, acc = flash_update(m, l, acc, kc, vc)

        out_ref[:, :] = acc / l

        for r in z_rdmas + p_rdmas:
            r.wait_send()

    return pl.pallas_call(
        body,
        out_shape=jax.ShapeDtypeStruct((s_per, d), jnp.float32),
        in_specs=[
            pl.BlockSpec(memory_space=pltpu.VMEM),
            pl.BlockSpec(memory_space=pltpu.VMEM),
            pl.BlockSpec(memory_space=pltpu.VMEM),
        ],
        out_specs=pl.BlockSpec(memory_space=pltpu.VMEM),
        scratch_shapes=[
            pltpu.VMEM((PG, ZG, 2, s_per, d), jnp.bfloat16),
            pltpu.SemaphoreType.DMA((ZG,)),
            pltpu.SemaphoreType.DMA((ZG,)),
            pltpu.SemaphoreType.DMA((PG,)),
            pltpu.SemaphoreType.DMA((PG,)),
        ],
        compiler_params=pltpu.CompilerParams(collective_id=0),
    )(q, k, v)
